# Initial kernel scaffold; baseline (speedup 1.0000x reference)
import jax
import jax.numpy as jnp
from jax.experimental import pallas as pl


def kernel(imgs, ln1_g, W_lin, b_lin, ln2_g, pos_embed_height, pos_embed_width):
    raise NotImplementedError("write your pallas kernel here")


# speedup vs baseline: 1.2756x; 1.2756x over previous
import jax
import jax.numpy as jnp
from jax.experimental import pallas as pl

PATCH = 16
EPS = 1e-5


def _embed_kernel(tok_ref, wt_ref, blin_ref, ln1g_ref, ln2g_ref,
                  peh_ref, pew_ref, out_ref):
    x = tok_ref[0]
    m = jnp.mean(x, axis=-1, keepdims=True)
    xm = x - m
    v = jnp.mean(xm * xm, axis=-1, keepdims=True)
    x = xm * jax.lax.rsqrt(v + EPS) * ln1g_ref[0]
    y = jnp.dot(x, wt_ref[...], preferred_element_type=jnp.float32,
                precision=jax.lax.Precision.HIGHEST)
    y = y + blin_ref[0]
    m2 = jnp.mean(y, axis=-1, keepdims=True)
    ym = y - m2
    v2 = jnp.mean(ym * ym, axis=-1, keepdims=True)
    y = ym * jax.lax.rsqrt(v2 + EPS) * ln2g_ref[0]
    pos = peh_ref[...][:, None, :] + pew_ref[...][None, :, :]
    out_ref[0] = y + pos.reshape(out_ref.shape[1], out_ref.shape[2])


def kernel(imgs, ln1_g, W_lin, b_lin, ln2_g, pos_embed_height, pos_embed_width):
    B, C, H, W = imgs.shape
    P = PATCH
    h, w = H // P, W // P
    dim, patch_dim = W_lin.shape
    n_tok = h * w

    tokens = imgs.reshape(B, C, h, P, w, P).transpose(0, 2, 4, 1, 3, 5)
    tokens = tokens.reshape(B, n_tok, patch_dim)

    out = pl.pallas_call(
        _embed_kernel,
        grid=(B,),
        in_specs=[
            pl.BlockSpec((1, n_tok, patch_dim), lambda i: (i, 0, 0)),
            pl.BlockSpec((patch_dim, dim), lambda i: (0, 0)),
            pl.BlockSpec((1, dim), lambda i: (0, 0)),
            pl.BlockSpec((1, patch_dim), lambda i: (0, 0)),
            pl.BlockSpec((1, dim), lambda i: (0, 0)),
            pl.BlockSpec((h, dim), lambda i: (0, 0)),
            pl.BlockSpec((w, dim), lambda i: (0, 0)),
        ],
        out_specs=pl.BlockSpec((1, n_tok, dim), lambda i: (i, 0, 0)),
        out_shape=jax.ShapeDtypeStruct((B, n_tok, dim), jnp.float32),
    )(tokens, W_lin.T, b_lin.reshape(1, dim), ln1_g.reshape(1, patch_dim),
      ln2_g.reshape(1, dim), pos_embed_height, pos_embed_width)
    return out


# speedup vs baseline: 1.5003x; 1.1762x over previous
import jax
import jax.numpy as jnp
from jax.experimental import pallas as pl

PATCH = 16
EPS = 1e-5


def _embed_kernel(tok_ref, wt_ref, blin_ref, ln1g_ref, ln2g_ref,
                  peh_ref, pew_ref, out_ref):
    x = tok_ref[0]
    m = jnp.mean(x, axis=-1, keepdims=True)
    xm = x - m
    v = jnp.mean(xm * xm, axis=-1, keepdims=True)
    x = xm * jax.lax.rsqrt(v + EPS) * ln1g_ref[0]
    y = jnp.dot(x, wt_ref[...], preferred_element_type=jnp.float32)
    y = y + blin_ref[0]
    m2 = jnp.mean(y, axis=-1, keepdims=True)
    ym = y - m2
    v2 = jnp.mean(ym * ym, axis=-1, keepdims=True)
    y = ym * jax.lax.rsqrt(v2 + EPS) * ln2g_ref[0]
    pos = peh_ref[...][:, None, :] + pew_ref[...][None, :, :]
    out_ref[0] = y + pos.reshape(out_ref.shape[1], out_ref.shape[2])


def kernel(imgs, ln1_g, W_lin, b_lin, ln2_g, pos_embed_height, pos_embed_width):
    B, C, H, W = imgs.shape
    P = PATCH
    h, w = H // P, W // P
    dim, patch_dim = W_lin.shape
    n_tok = h * w

    tokens = imgs.reshape(B, C, h, P, w, P).transpose(0, 2, 4, 1, 3, 5)
    tokens = tokens.reshape(B, n_tok, patch_dim)

    out = pl.pallas_call(
        _embed_kernel,
        grid=(B,),
        in_specs=[
            pl.BlockSpec((1, n_tok, patch_dim), lambda i: (i, 0, 0)),
            pl.BlockSpec((patch_dim, dim), lambda i: (0, 0)),
            pl.BlockSpec((1, dim), lambda i: (0, 0)),
            pl.BlockSpec((1, patch_dim), lambda i: (0, 0)),
            pl.BlockSpec((1, dim), lambda i: (0, 0)),
            pl.BlockSpec((h, dim), lambda i: (0, 0)),
            pl.BlockSpec((w, dim), lambda i: (0, 0)),
        ],
        out_specs=pl.BlockSpec((1, n_tok, dim), lambda i: (i, 0, 0)),
        out_shape=jax.ShapeDtypeStruct((B, n_tok, dim), jnp.float32),
    )(tokens, W_lin.T, b_lin.reshape(1, dim), ln1_g.reshape(1, patch_dim),
      ln2_g.reshape(1, dim), pos_embed_height, pos_embed_width)
    return out


# speedup vs baseline: 2.5481x; 1.6984x over previous
import jax
import jax.numpy as jnp
from jax.experimental import pallas as pl

PATCH = 16
EPS = 1e-5


def _embed_kernel(img_ref, wt_ref, blin_ref, ln1g_ref, ln2g_ref,
                  peh_ref, pew_ref, out_ref):
    C, H, W = img_ref.shape[1:]
    P = PATCH
    h, w = H // P, W // P
    im = img_ref[0]
    x = im.reshape(C, h, P, w, P).transpose(1, 3, 0, 2, 4).reshape(h * w, C * P * P)
    m = jnp.mean(x, axis=-1, keepdims=True)
    xm = x - m
    v = jnp.mean(xm * xm, axis=-1, keepdims=True)
    x = xm * jax.lax.rsqrt(v + EPS) * ln1g_ref[0]
    y = jnp.dot(x, wt_ref[...], preferred_element_type=jnp.float32)
    y = y + blin_ref[0]
    m2 = jnp.mean(y, axis=-1, keepdims=True)
    ym = y - m2
    v2 = jnp.mean(ym * ym, axis=-1, keepdims=True)
    y = ym * jax.lax.rsqrt(v2 + EPS) * ln2g_ref[0]
    pos = peh_ref[...][:, None, :] + pew_ref[...][None, :, :]
    out_ref[0] = y + pos.reshape(out_ref.shape[1], out_ref.shape[2])


def kernel(imgs, ln1_g, W_lin, b_lin, ln2_g, pos_embed_height, pos_embed_width):
    B, C, H, W = imgs.shape
    P = PATCH
    h, w = H // P, W // P
    dim, patch_dim = W_lin.shape
    n_tok = h * w

    out = pl.pallas_call(
        _embed_kernel,
        grid=(B,),
        in_specs=[
            pl.BlockSpec((1, C, H, W), lambda i: (i, 0, 0, 0)),
            pl.BlockSpec((patch_dim, dim), lambda i: (0, 0)),
            pl.BlockSpec((1, dim), lambda i: (0, 0)),
            pl.BlockSpec((1, patch_dim), lambda i: (0, 0)),
            pl.BlockSpec((1, dim), lambda i: (0, 0)),
            pl.BlockSpec((h, dim), lambda i: (0, 0)),
            pl.BlockSpec((w, dim), lambda i: (0, 0)),
        ],
        out_specs=pl.BlockSpec((1, n_tok, dim), lambda i: (i, 0, 0)),
        out_shape=jax.ShapeDtypeStruct((B, n_tok, dim), jnp.float32),
    )(imgs, W_lin.T, b_lin.reshape(1, dim), ln1_g.reshape(1, patch_dim),
      ln2_g.reshape(1, dim), pos_embed_height, pos_embed_width)
    return out


# speedup vs baseline: 2.6272x; 1.0310x over previous
import jax
import jax.numpy as jnp
from jax.experimental import pallas as pl

PATCH = 16
EPS = 1e-5


def _embed_kernel(img_ref, wt_ref, blin_ref, ln1g_ref, ln2g_ref,
                  peh_ref, pew_ref, out_ref):
    C, H, W = img_ref.shape[1:]
    P = PATCH
    h, w = H // P, W // P
    im = img_ref[0]
    x = im.reshape(C, h, P, w, P).transpose(1, 3, 0, 2, 4).reshape(h * w, C * P * P)
    m = jnp.mean(x, axis=-1, keepdims=True)
    xm = x - m
    v = jnp.mean(xm * xm, axis=-1, keepdims=True)
    x = xm * jax.lax.rsqrt(v + EPS) * ln1g_ref[0]
    y = jnp.dot(x, wt_ref[...], preferred_element_type=jnp.float32)
    y = y + blin_ref[0]
    m2 = jnp.mean(y, axis=-1, keepdims=True)
    ym = y - m2
    v2 = jnp.mean(ym * ym, axis=-1, keepdims=True)
    y = ym * jax.lax.rsqrt(v2 + EPS) * ln2g_ref[0]
    pos = peh_ref[...][:, None, :] + pew_ref[...][None, :, :]
    out_ref[0] = y + pos.reshape(out_ref.shape[1], out_ref.shape[2])


def kernel(imgs, ln1_g, W_lin, b_lin, ln2_g, pos_embed_height, pos_embed_width):
    B, C, H, W = imgs.shape
    P = PATCH
    h, w = H // P, W // P
    dim, patch_dim = W_lin.shape
    n_tok = h * w

    SPLIT = 2
    hs = h // SPLIT

    out = pl.pallas_call(
        _embed_kernel,
        grid=(B, SPLIT),
        in_specs=[
            pl.BlockSpec((1, C, H // SPLIT, W), lambda i, j: (i, 0, j, 0)),
            pl.BlockSpec((patch_dim, dim), lambda i, j: (0, 0)),
            pl.BlockSpec((1, dim), lambda i, j: (0, 0)),
            pl.BlockSpec((1, patch_dim), lambda i, j: (0, 0)),
            pl.BlockSpec((1, dim), lambda i, j: (0, 0)),
            pl.BlockSpec((hs, dim), lambda i, j: (j, 0)),
            pl.BlockSpec((w, dim), lambda i, j: (0, 0)),
        ],
        out_specs=pl.BlockSpec((1, hs * w, dim), lambda i, j: (i, j, 0)),
        out_shape=jax.ShapeDtypeStruct((B, n_tok, dim), jnp.float32),
    )(imgs, W_lin.T, b_lin.reshape(1, dim), ln1_g.reshape(1, patch_dim),
      ln2_g.reshape(1, dim), pos_embed_height, pos_embed_width)
    return out
